# baseline (speedup 1.0000x reference)
import functools

import jax
import jax.numpy as jnp
from jax import lax
from jax.experimental import pallas as pl
from jax.experimental.pallas import tpu as pltpu
from jax.experimental.pallas import tpu_sc as plsc

N_FACTORS = 32
BATCH = 16384

NUM_CORES = 2
NUM_SUBCORES = 16
LANES = 16
NUM_WORKERS = NUM_CORES * NUM_SUBCORES
B_PER_W = BATCH // NUM_WORKERS
IDX_CHUNK = 128
N_CHUNKS = B_PER_W // IDX_CHUNK
N_GROUPS = B_PER_W // LANES

_mesh = plsc.VectorSubcoreMesh(
    core_axis_name="c", subcore_axis_name="s",
    num_cores=NUM_CORES, num_subcores=NUM_SUBCORES,
)


@functools.partial(
    pl.kernel,
    out_type=jax.ShapeDtypeStruct((BATCH,), jnp.float32),
    mesh=_mesh,
    compiler_params=pltpu.CompilerParams(
        needs_layout_passes=False, use_tc_tiling_on_sc=False),
    scratch_types=dict(
        uidx=pltpu.VMEM((N_CHUNKS, IDX_CHUNK), jnp.int32),
        midx=pltpu.VMEM((N_CHUNKS, IDX_CHUNK), jnp.int32),
        ucols=pltpu.VMEM((N_FACTORS, B_PER_W), jnp.float32),
        mcols=pltpu.VMEM((N_FACTORS, B_PER_W), jnp.float32),
        out_v=pltpu.VMEM((B_PER_W,), jnp.float32),
        sem=pltpu.SemaphoreType.DMA,
    ),
)
def _sc_body(user_ids, movie_ids, uft, mft, out_hbm,
             uidx, midx, ucols, mcols, out_v, sem):
    wid = lax.axis_index("s") * NUM_CORES + lax.axis_index("c")
    base = wid * B_PER_W

    for c in range(N_CHUNKS):
        off = base + c * IDX_CHUNK
        pltpu.sync_copy(user_ids.at[pl.ds(off, IDX_CHUNK)], uidx.at[c])
        pltpu.sync_copy(movie_ids.at[pl.ds(off, IDX_CHUNK)], midx.at[c])

    copies = []
    for f in range(N_FACTORS):
        for c in range(N_CHUNKS):
            dst = ucols.at[f, pl.ds(c * IDX_CHUNK, IDX_CHUNK)]
            copies.append(pltpu.async_copy(uft.at[f].at[uidx.at[c]], dst, sem))
            dst = mcols.at[f, pl.ds(c * IDX_CHUNK, IDX_CHUNK)]
            copies.append(pltpu.async_copy(mft.at[f].at[midx.at[c]], dst, sem))
    for cp in copies:
        cp.wait()

    def group_body(g, _):
        sl = pl.ds(g * LANES, LANES)
        acc = jnp.zeros((LANES,), jnp.float32)
        for f in range(N_FACTORS):
            acc = acc + ucols[f, sl] * mcols[f, sl]
        out_v[sl] = acc
        return 0

    lax.fori_loop(0, N_GROUPS, group_body, 0)

    pltpu.sync_copy(out_v, out_hbm.at[pl.ds(base, B_PER_W)])


def kernel(user_ids, movie_ids, user_factors, movie_factors):
    out = _sc_body(
        user_ids.astype(jnp.int32),
        movie_ids.astype(jnp.int32),
        user_factors.T,
        movie_factors.T,
    )
    return out.reshape(-1, 1)
